# Initial kernel scaffold; baseline (speedup 1.0000x reference)
import jax
import jax.numpy as jnp
from jax.experimental import pallas as pl


def kernel(x, W0, b0, W1, b1, W2, b2, W3, b3, W4, b4, W5, b5, labels):
    raise NotImplementedError("write your pallas kernel here")


# speedup vs baseline: 1.0820x; 1.0820x over previous
import functools

import jax
import jax.numpy as jnp
from jax import lax
from jax.experimental import pallas as pl
from jax.experimental.pallas import tpu as pltpu

LEVEL_SIZES_K = [6, 24, 96, 384, 1536, 6144]
NUM_LABELS_K = 8190
IN_FEAT = 768
K_AUG = 3200
N_PAD = 8192
SEG_OFF = [0, 768, 896, 1024, 1152, 1536]
SEG_W = [768, 6, 24, 96, 384, 1536]
ONES_COL = 3072
Y0_OFF = 3073


def _phase_a_kernel(x_ref, w0t, b0, w1t, b1, w2t, b2, w3t, b3, w4t, b4,
                    out_ref):
    bb = x_ref.shape[0]
    out_ref[...] = jnp.zeros((bb, K_AUG), jnp.float32)
    x = x_ref[...]
    out_ref[:, 0:768] = jnp.maximum(x, 0.0)
    y0 = jnp.dot(x, w0t[...], preferred_element_type=jnp.float32) + b0[...]
    out_ref[:, 768:774] = jnp.maximum(y0, 0.0)
    out_ref[:, Y0_OFF:Y0_OFF + 6] = y0
    out_ref[:, ONES_COL:ONES_COL + 1] = jnp.ones((bb, 1), jnp.float32)
    for lvl, (wt, b) in enumerate(((w1t, b1), (w2t, b2), (w3t, b3),
                                   (w4t, b4)), start=1):
        k_in = wt.shape[0]
        y = jnp.dot(out_ref[:, 0:k_in], wt[...],
                    preferred_element_type=jnp.float32) + b[...]
        o = SEG_OFF[lvl + 1]
        out_ref[:, o:o + SEG_W[lvl + 1]] = jnp.maximum(y, 0.0)


def _gather_kernel(idx_ref, wsrc_ref, out_ref, sem):
    step = pl.program_id(0)
    rows_per_step = out_ref.shape[0] // pl.num_programs(0)
    base = step * rows_per_step

    def issue(i, _):
        j = base + i
        src = idx_ref[j]
        pltpu.make_async_copy(
            wsrc_ref.at[pl.ds(src, 1), :],
            out_ref.at[pl.ds(j, 1), :],
            sem,
        ).start()
        return 0

    lax.fori_loop(0, rows_per_step, issue, 0)
    pltpu.make_async_copy(
        wsrc_ref.at[pl.ds(0, rows_per_step), :],
        out_ref.at[pl.ds(base, rows_per_step), :],
        sem,
    ).wait()


def _matmul_kernel(cur_ref, w_ref, out_ref):
    out_ref[...] = lax.dot_general(
        cur_ref[...], w_ref[...],
        dimension_numbers=(((1,), (1,)), ((), ())),
        preferred_element_type=jnp.float32)


def kernel(x, W0, b0, W1, b1, W2, b2, W3, b3, W4, b4, W5, b5, labels):
    batch = x.shape[0]
    f32 = jnp.float32
    Ws = [W0, W1, W2, W3, W4, W5]
    bs = [b0, b1, b2, b3, b4, b5]

    wts = []
    for lvl in range(5):
        w = Ws[lvl]
        parts = []
        col = 0
        for s in range(lvl + 1):
            seg = w[:, col:col + SEG_W[s]]
            col += SEG_W[s]
            pad_to = SEG_OFF[s + 1] - SEG_OFF[s]
            if pad_to > SEG_W[s]:
                seg = jnp.pad(seg, ((0, 0), (0, pad_to - SEG_W[s])))
            parts.append(seg)
        wp = jnp.concatenate(parts, axis=1) if len(parts) > 1 else parts[0]
        wts.append(wp.T)
    brs = [b.reshape(1, -1) for b in bs]

    blocks = []
    lvl0 = jnp.zeros((6, K_AUG), f32).at[:, Y0_OFF:Y0_OFF + 6].set(
        jnp.eye(6, dtype=f32))
    blocks.append(lvl0)
    for lvl in range(1, 6):
        w = Ws[lvl]
        parts = [jnp.zeros((w.shape[0], K_AUG), f32)]
        row = parts[0]
        col = 0
        for s in range(lvl + 1):
            seg = w[:, col:col + SEG_W[s]]
            col += SEG_W[s]
            row = row.at[:, SEG_OFF[s]:SEG_OFF[s] + SEG_W[s]].set(seg)
        row = row.at[:, ONES_COL].set(bs[lvl])
        blocks.append(row)
    blocks.append(jnp.zeros((2, K_AUG), f32))
    wsrc = jnp.concatenate(blocks, axis=0)

    labels_i = labels.astype(jnp.int32)
    inv = jnp.zeros((NUM_LABELS_K,), jnp.int32).at[labels_i].set(
        jnp.arange(NUM_LABELS_K, dtype=jnp.int32))
    inv_ext = jnp.concatenate(
        [inv, jnp.array([NUM_LABELS_K, NUM_LABELS_K + 1], jnp.int32)])

    bb = 256
    grid1 = (batch // bb,)
    cur = pl.pallas_call(
        _phase_a_kernel,
        grid=grid1,
        in_specs=[pl.BlockSpec((bb, IN_FEAT), lambda i: (i, 0))] + [
            spec for lvl in range(5) for spec in (
                pl.BlockSpec(wts[lvl].shape, lambda i: (0, 0)),
                pl.BlockSpec(brs[lvl].shape, lambda i: (0, 0)),
            )
        ],
        out_specs=pl.BlockSpec((bb, K_AUG), lambda i: (i, 0)),
        out_shape=jax.ShapeDtypeStruct((batch, K_AUG), f32),
        compiler_params=pltpu.CompilerParams(
            dimension_semantics=("parallel",),
            vmem_limit_bytes=56 * 1024 * 1024,
        ),
    )(x, wts[0], brs[0], wts[1], brs[1], wts[2], brs[2], wts[3], brs[3],
      wts[4], brs[4])

    wbig = pl.pallas_call(
        _gather_kernel,
        grid=(16,),
        in_specs=[
            pl.BlockSpec(memory_space=pltpu.SMEM),
            pl.BlockSpec(memory_space=pl.ANY),
        ],
        out_specs=pl.BlockSpec(memory_space=pl.ANY),
        out_shape=jax.ShapeDtypeStruct((N_PAD, K_AUG), f32),
        scratch_shapes=[pltpu.SemaphoreType.DMA],
        compiler_params=pltpu.CompilerParams(
            dimension_semantics=("parallel",),
        ),
    )(inv_ext, wsrc)

    bm, bn = 512, 1024
    grid3 = (N_PAD // bn, batch // bm)
    logits = pl.pallas_call(
        _matmul_kernel,
        grid=grid3,
        in_specs=[
            pl.BlockSpec((bm, K_AUG), lambda c, b: (b, 0)),
            pl.BlockSpec((bn, K_AUG), lambda c, b: (c, 0)),
        ],
        out_specs=pl.BlockSpec((bm, bn), lambda c, b: (b, c)),
        out_shape=jax.ShapeDtypeStruct((batch, N_PAD), f32),
        compiler_params=pltpu.CompilerParams(
            dimension_semantics=("parallel", "arbitrary"),
            vmem_limit_bytes=56 * 1024 * 1024,
        ),
    )(cur, wbig)

    return logits[:, :NUM_LABELS_K]
